# baseline (speedup 1.0000x reference)
import functools
import jax
import jax.numpy as jnp
from jax import lax
from jax.experimental import pallas as pl
from jax.experimental.pallas import tpu as pltpu
from jax.experimental.pallas import tpu_sc as plsc

_NC = 2
_NS = 16
_L = 16


def _tc_block(v_ref, o_ref, *, lanes, n_rows):
    c = pl.program_id(1)
    x = v_ref[0]
    xm1 = jnp.roll(x, -1, axis=0)
    xm2 = jnp.roll(x, -2, axis=0)
    xm3 = jnp.roll(x, -3, axis=0)
    row = jax.lax.broadcasted_iota(jnp.int32, (n_rows, 1), 0)
    even = (row & 1) == 0
    z = jnp.where(even, jnp.minimum(xm2, xm3), jnp.maximum(xm2, xm1))

    lane = jax.lax.broadcasted_iota(jnp.int32, (1, lanes), 1)
    nbits = max(1, (lanes - 1).bit_length())
    for k in range(nbits):
        amt = 2 << k
        bit = ((lane >> k) & 1) == 1
        z = jnp.where(bit, jnp.roll(z, amt, axis=0), z)

    z = jax.lax.cond(
        c == 0,
        lambda t: jnp.where(lane == 0, jnp.roll(t, 2, axis=0), t),
        lambda t: t,
        z)

    ch = 2 * lanes
    base = 2 * lanes * c
    for p in range(0, n_rows, ch):
        row0 = (p + base) % n_rows
        o_ref[0, pl.ds(row0, ch), :] = z[p:p + ch, :]


def _sc_body(vpad, out, xa, xb, ya, yb, sia, sib, soa, sob,
             *, n, d_sc, nb, r_pairs, grp_off):
    half = n // 2
    ngrp = d_sc // _L
    ntask = nb * ngrp
    nw = _NC * _NS
    per_w = ntask // nw
    nchunk = half // r_pairs

    wid = lax.axis_index("s") * _NC + lax.axis_index("c")
    lane = lax.iota(jnp.int32, _L)

    xs = (xa, xb)
    ys = (ya, yb)
    sis = (sia, sib)
    sos = (soa, sob)

    spw = ntask * nchunk // nw
    steps = list(range(spw))

    def step_bgm(s):
        gs = wid * spw + s
        task = gs // nchunk
        m = gs % nchunk
        return task // ngrp, task % ngrp, m

    def in_src(s):
        b, g, m = step_bgm(s)
        a = m * r_pairs
        w0 = (a - (grp_off + g) * _L - (_L - 1) + 2 * half) % half
        return vpad.at[b, pl.ds(2 * w0, 2 * (r_pairs + _L)),
                       pl.ds(g * _L, _L)]

    def out_dst(s):
        b, g, m = step_bgm(s)
        a = m * r_pairs
        return out.at[b, pl.ds(2 * a, 2 * r_pairs), pl.ds(g * _L, _L)]

    in_flight = {}
    out_flight = {}
    in_flight[0] = pltpu.async_copy(in_src(0), xs[0], sis[0])

    for s in steps:
        p = s % 2
        if s + 1 < len(steps):
            in_flight[s + 1] = pltpu.async_copy(
                in_src(s + 1), xs[1 - p], sis[1 - p])
        in_flight.pop(s).wait()
        if s >= 2:
            out_flight.pop(s - 2).wait()

        base_a = 2 * _L - 2 * lane
        x_ref = xs[p]
        y_ref = ys[p]

        def body(kr, row_a):
            va = plsc.load_gather(x_ref, [row_a, lane])
            vb = plsc.load_gather(x_ref, [row_a + 1, lane])
            y_ref[2 * kr] = jnp.minimum(va, vb)
            y_ref[2 * kr + 1] = jnp.maximum(va, vb)
            return row_a + 2

        lax.fori_loop(0, r_pairs, body, base_a, unroll=16)

        out_flight[s] = pltpu.async_copy(y_ref, out_dst(s), sos[p])

    out_flight.pop(len(steps) - 2).wait()
    out_flight.pop(len(steps) - 1).wait()


def kernel(v):
    nb, n, d = v.shape
    lanes = 128
    d_tc = 768
    d_sc = d - d_tc

    tc = pl.pallas_call(
        functools.partial(_tc_block, lanes=lanes, n_rows=n),
        grid=(nb, d_tc // lanes),
        in_specs=[pl.BlockSpec((1, n, lanes), lambda i, j: (i, 0, j))],
        out_specs=pl.BlockSpec((1, n, lanes), lambda i, j: (i, 0, j)),
        out_shape=jax.ShapeDtypeStruct((nb, n, d), v.dtype),
    )(v)

    r_pairs = 512
    pad = 2 * (r_pairs + _L)
    v_sc = v[:, :, d_tc:]
    vpad = jnp.concatenate([v_sc, v_sc[:, :pad, :]], axis=1)
    mesh = plsc.VectorSubcoreMesh(core_axis_name="c", subcore_axis_name="s",
                                  num_cores=_NC, num_subcores=_NS)
    sc = pl.kernel(
        functools.partial(_sc_body, n=n, d_sc=d_sc, nb=nb, r_pairs=r_pairs,
                          grp_off=d_tc // _L),
        out_type=jax.ShapeDtypeStruct((nb, n, d_sc), v.dtype),
        mesh=mesh,
        compiler_params=pltpu.CompilerParams(use_tc_tiling_on_sc=False,
                                             needs_layout_passes=False),
        scratch_types=[
            pltpu.VMEM((2 * (r_pairs + _L), _L), jnp.float32),
            pltpu.VMEM((2 * (r_pairs + _L), _L), jnp.float32),
            pltpu.VMEM((2 * r_pairs, _L), jnp.float32),
            pltpu.VMEM((2 * r_pairs, _L), jnp.float32),
            pltpu.SemaphoreType.DMA,
            pltpu.SemaphoreType.DMA,
            pltpu.SemaphoreType.DMA,
            pltpu.SemaphoreType.DMA,
        ],
    )(vpad)

    return jax.lax.dynamic_update_slice(tc, sc, (0, 0, d_tc))


# speedup vs baseline: 1.2767x; 1.2767x over previous
import functools
import jax
import jax.numpy as jnp
from jax import lax
from jax.experimental import pallas as pl
from jax.experimental.pallas import tpu as pltpu
from jax.experimental.pallas import tpu_sc as plsc

_NC = 2
_NS = 16
_L = 16


def _tc_block(v_ref, o_ref, *, lanes, n_rows):
    c = pl.program_id(1)
    x = v_ref[0]
    xm1 = jnp.roll(x, -1, axis=0)
    xm2 = jnp.roll(x, -2, axis=0)
    xm3 = jnp.roll(x, -3, axis=0)
    row = jax.lax.broadcasted_iota(jnp.int32, (n_rows, 1), 0)
    even = (row & 1) == 0
    z = jnp.where(even, jnp.minimum(xm2, xm3), jnp.maximum(xm2, xm1))

    lane = jax.lax.broadcasted_iota(jnp.int32, (1, lanes), 1)
    nbits = max(1, (lanes - 1).bit_length())
    for k in range(nbits):
        amt = 2 << k
        bit = ((lane >> k) & 1) == 1
        z = jnp.where(bit, jnp.roll(z, amt, axis=0), z)

    z = jax.lax.cond(
        c == 0,
        lambda t: jnp.where(lane == 0, jnp.roll(t, 2, axis=0), t),
        lambda t: t,
        z)

    ch = 2 * lanes
    base = 2 * lanes * c
    for p in range(0, n_rows, ch):
        row0 = (p + base) % n_rows
        o_ref[0, pl.ds(row0, ch), :] = z[p:p + ch, :]


def _sc_body(vpad, out, xa, xb, ya, yb, sia, sib, soa, sob,
             *, n, d_sc, nb, r_pairs, grp_off):
    half = n // 2
    ngrp = d_sc // _L
    ntask = nb * ngrp
    nw = _NC * _NS
    per_w = ntask // nw
    nchunk = half // r_pairs

    wid = lax.axis_index("s") * _NC + lax.axis_index("c")
    lane = lax.iota(jnp.int32, _L)

    xs = (xa, xb)
    ys = (ya, yb)
    sis = (sia, sib)
    sos = (soa, sob)

    spw = ntask * nchunk // nw
    steps = list(range(spw))

    def step_bgm(s):
        gs = wid * spw + s
        task = gs // nchunk
        m = gs % nchunk
        return task // ngrp, task % ngrp, m

    def in_src(s):
        b, g, m = step_bgm(s)
        a = m * r_pairs
        w0 = (a - (grp_off + g) * _L - (_L - 1) + 2 * half) % half
        return vpad.at[b, pl.ds(2 * w0, 2 * (r_pairs + _L)),
                       pl.ds(g * _L, _L)]

    def out_dst(s):
        b, g, m = step_bgm(s)
        a = m * r_pairs
        return out.at[b, pl.ds(2 * a, 2 * r_pairs), pl.ds(g * _L, _L)]

    in_flight = {}
    out_flight = {}
    in_flight[0] = pltpu.async_copy(in_src(0), xs[0], sis[0])

    for s in steps:
        p = s % 2
        if s + 1 < len(steps):
            in_flight[s + 1] = pltpu.async_copy(
                in_src(s + 1), xs[1 - p], sis[1 - p])
        in_flight.pop(s).wait()
        if s >= 2:
            out_flight.pop(s - 2).wait()

        base_a = 2 * _L - 2 * lane
        x_ref = xs[p]
        y_ref = ys[p]

        def body(kr, row_a):
            va = plsc.load_gather(x_ref, [row_a, lane])
            vb = plsc.load_gather(x_ref, [row_a + 1, lane])
            y_ref[2 * kr] = jnp.minimum(va, vb)
            y_ref[2 * kr + 1] = jnp.maximum(va, vb)
            return row_a + 2

        lax.fori_loop(0, r_pairs, body, base_a, unroll=16)

        out_flight[s] = pltpu.async_copy(y_ref, out_dst(s), sos[p])

    out_flight.pop(len(steps) - 2).wait()
    out_flight.pop(len(steps) - 1).wait()


def kernel(v):
    nb, n, d = v.shape
    lanes = 128
    d_tc = 896
    d_sc = d - d_tc

    tc = pl.pallas_call(
        functools.partial(_tc_block, lanes=lanes, n_rows=n),
        grid=(nb, d_tc // lanes),
        in_specs=[pl.BlockSpec((1, n, lanes), lambda i, j: (i, 0, j))],
        out_specs=pl.BlockSpec((1, n, lanes), lambda i, j: (i, 0, j)),
        out_shape=jax.ShapeDtypeStruct((nb, n, d), v.dtype),
    )(v)

    r_pairs = 512
    pad = 2 * (r_pairs + _L)
    v_sc = v[:, :, d_tc:]
    vpad = jnp.concatenate([v_sc, v_sc[:, :pad, :]], axis=1)
    mesh = plsc.VectorSubcoreMesh(core_axis_name="c", subcore_axis_name="s",
                                  num_cores=_NC, num_subcores=_NS)
    sc = pl.kernel(
        functools.partial(_sc_body, n=n, d_sc=d_sc, nb=nb, r_pairs=r_pairs,
                          grp_off=d_tc // _L),
        out_type=jax.ShapeDtypeStruct((nb, n, d_sc), v.dtype),
        mesh=mesh,
        compiler_params=pltpu.CompilerParams(use_tc_tiling_on_sc=False,
                                             needs_layout_passes=False),
        scratch_types=[
            pltpu.VMEM((2 * (r_pairs + _L), _L), jnp.float32),
            pltpu.VMEM((2 * (r_pairs + _L), _L), jnp.float32),
            pltpu.VMEM((2 * r_pairs, _L), jnp.float32),
            pltpu.VMEM((2 * r_pairs, _L), jnp.float32),
            pltpu.SemaphoreType.DMA,
            pltpu.SemaphoreType.DMA,
            pltpu.SemaphoreType.DMA,
            pltpu.SemaphoreType.DMA,
        ],
    )(vpad)

    return jax.lax.dynamic_update_slice(tc, sc, (0, 0, d_tc))
